# baseline (speedup 1.0000x reference)
import jax
import jax.numpy as jnp
from jax import lax
from jax.experimental import pallas as pl
from jax.experimental.pallas import tpu as pltpu

NUM_GRAPHS = 256
ROW_BLOCK = 2000


def _pool_body(x_ref, batch_ref, w1_ref, b1_ref, w2_ref, out_ref,
               acc_num, acc_den):
    i = pl.program_id(0)
    nb = pl.num_programs(0)

    @pl.when(i == 0)
    def _init():
        acc_num[...] = jnp.zeros_like(acc_num)
        acc_den[...] = jnp.zeros_like(acc_den)

    x = x_ref[...]
    h = jnp.tanh(
        jnp.dot(x, w1_ref[...], preferred_element_type=jnp.float32)
        + b1_ref[...])
    s = lax.dot_general(h, w2_ref[...], (((1,), (1,)), ((), ())),
                        preferred_element_type=jnp.float32)
    e = jnp.exp(s)

    batch_row = batch_ref[0, :, :]
    r = batch_row.shape[1]
    segT = lax.broadcasted_iota(jnp.int32, (NUM_GRAPHS, r), 0)
    onehotT = jnp.where(batch_row == segT, 1.0, 0.0)

    m = x * e
    acc_num[...] += jnp.dot(onehotT, m,
                            preferred_element_type=jnp.float32)
    acc_den[...] += jnp.dot(onehotT, e,
                            preferred_element_type=jnp.float32)

    @pl.when(i == nb - 1)
    def _finish():
        den = acc_den[...]
        den = jnp.where(den > 0.0, den, 1.0)
        out_ref[...] = acc_num[...] / den


def _pool_one_type(x, batch, W1, b1, W2, row_block=ROW_BLOCK):
    n = x.shape[0]
    assert n % row_block == 0, n
    nb = n // row_block
    batch3 = batch.reshape(nb, 1, row_block)
    return pl.pallas_call(
        _pool_body,
        grid=(nb,),
        in_specs=[
            pl.BlockSpec((row_block, 128), lambda i: (i, 0)),
            pl.BlockSpec((1, 1, row_block), lambda i: (i, 0, 0)),
            pl.BlockSpec((128, 64), lambda i: (0, 0)),
            pl.BlockSpec((1, 64), lambda i: (0, 0)),
            pl.BlockSpec((1, 64), lambda i: (0, 0)),
        ],
        out_specs=pl.BlockSpec((NUM_GRAPHS, 128), lambda i: (0, 0)),
        out_shape=jax.ShapeDtypeStruct((NUM_GRAPHS, 128), jnp.float32),
        scratch_shapes=[
            pltpu.VMEM((NUM_GRAPHS, 128), jnp.float32),
            pltpu.VMEM((NUM_GRAPHS, 1), jnp.float32),
        ],
        compiler_params=pltpu.CompilerParams(
            dimension_semantics=("arbitrary",)),
    )(x, batch3, W1, b1.reshape(1, 64), W2.reshape(1, 64))


def kernel(x_type0, x_type1, batch_type0, batch_type1, W1, b1, W2, b2):
    del b2
    pooled0 = _pool_one_type(x_type0, batch_type0, W1, b1, W2, row_block=25000)
    pooled1 = _pool_one_type(x_type1, batch_type1, W1, b1, W2, row_block=10000)
    return (pooled0 + pooled1) * 0.5


# speedup vs baseline: 1.0709x; 1.0709x over previous
import jax
import jax.numpy as jnp
from jax import lax
from jax.experimental import pallas as pl
from jax.experimental.pallas import tpu as pltpu

NUM_GRAPHS = 256
ROW_BLOCK = 2000


def _pool_body(x_ref, batch_ref, w1_ref, b1_ref, w2_ref, out_ref,
               acc_num, acc_den):
    i = pl.program_id(0)
    nb = pl.num_programs(0)

    @pl.when(i == 0)
    def _init():
        acc_num[...] = jnp.zeros_like(acc_num)
        acc_den[...] = jnp.zeros_like(acc_den)

    x = x_ref[...]
    h = jnp.tanh(
        jnp.dot(x, w1_ref[...], preferred_element_type=jnp.float32)
        + b1_ref[...])
    s = lax.dot_general(h, w2_ref[...], (((1,), (1,)), ((), ())),
                        preferred_element_type=jnp.float32)
    e = jnp.exp(s)

    batch_row = batch_ref[0, :, :]
    r = batch_row.shape[1]
    segT = lax.broadcasted_iota(jnp.int32, (NUM_GRAPHS, r), 0)
    onehotT = jnp.where(batch_row == segT, 1.0, 0.0)

    m = x * e
    acc_num[...] += jnp.dot(onehotT, m,
                            preferred_element_type=jnp.float32)
    acc_den[...] += jnp.dot(onehotT, e,
                            preferred_element_type=jnp.float32)

    @pl.when(i == nb - 1)
    def _finish():
        den = acc_den[...]
        den = jnp.where(den > 0.0, den, 1.0)
        out_ref[...] = acc_num[...] / den


def _pool_one_type(x, batch, W1, b1, W2, row_block=ROW_BLOCK):
    n = x.shape[0]
    assert n % row_block == 0, n
    nb = n // row_block
    batch3 = batch.reshape(nb, 1, row_block)
    return pl.pallas_call(
        _pool_body,
        grid=(nb,),
        in_specs=[
            pl.BlockSpec((row_block, 128), lambda i: (i, 0)),
            pl.BlockSpec((1, 1, row_block), lambda i: (i, 0, 0)),
            pl.BlockSpec((128, 64), lambda i: (0, 0)),
            pl.BlockSpec((1, 64), lambda i: (0, 0)),
            pl.BlockSpec((1, 64), lambda i: (0, 0)),
        ],
        out_specs=pl.BlockSpec((NUM_GRAPHS, 128), lambda i: (0, 0)),
        out_shape=jax.ShapeDtypeStruct((NUM_GRAPHS, 128), jnp.float32),
        scratch_shapes=[
            pltpu.VMEM((NUM_GRAPHS, 128), jnp.float32),
            pltpu.VMEM((NUM_GRAPHS, 1), jnp.float32),
        ],
        compiler_params=pltpu.CompilerParams(
            dimension_semantics=("arbitrary",)),
    )(x, batch3, W1, b1.reshape(1, 64), W2.reshape(1, 64))


def kernel(x_type0, x_type1, batch_type0, batch_type1, W1, b1, W2, b2):
    del b2
    pooled0 = _pool_one_type(x_type0, batch_type0, W1, b1, W2, row_block=20000)
    pooled1 = _pool_one_type(x_type1, batch_type1, W1, b1, W2, row_block=10000)
    return (pooled0 + pooled1) * 0.5
